# baseline (speedup 1.0000x reference)
import jax
import jax.numpy as jnp
from jax.experimental import pallas as pl
from jax.experimental.pallas import tpu as pltpu

_BS = 64
_NBUF = 16
_TG = 4
_TNBUF = 4


def _pipelined_kernel(x_hbm, t_hbm, o_hbm, xb, tb, ob, sx, st, so):
    seq_len, batch, _ = x_hbm.shape
    nsteps = seq_len // _BS
    ngroups = nsteps // _TG

    def x_copy(i):
        slot = i % _NBUF
        return pltpu.make_async_copy(
            x_hbm.at[pl.ds(i * _BS, _BS)], xb.at[slot], sx.at[slot])

    def t_copy(g):
        slot = g % _TNBUF
        return pltpu.make_async_copy(
            t_hbm.at[pl.ds(g * _TG * _BS, _TG * _BS)], tb.at[slot],
            st.at[slot])

    def out_copy(i):
        slot = i % _NBUF
        return pltpu.make_async_copy(
            ob.at[slot], o_hbm.at[pl.ds(i * _BS, _BS)], so.at[slot])

    for g in range(min(_TNBUF - 1, ngroups)):
        t_copy(g).start()
    for i in range(min(_NBUF, nsteps)):
        x_copy(i).start()

    for i in range(nsteps):
        slot = i % _NBUF
        g, r = divmod(i, _TG)
        if r == 0:
            t_copy(g).wait()
        x_copy(i).wait()
        if i >= _NBUF:
            out_copy(i - _NBUF).wait()
        t = tb[g % _TNBUF, pl.ds(r * _BS, _BS), :]
        for b in range(batch):
            ob[slot, :, b, :] = xb[slot, :, b, :] + t
        out_copy(i).start()
        if r == 0 and g + _TNBUF - 1 < ngroups:
            t_copy(g + _TNBUF - 1).start()
        if i + _NBUF < nsteps:
            x_copy(i + _NBUF).start()

    for i in range(max(0, nsteps - _NBUF), nsteps):
        out_copy(i).wait()


def kernel(x, table):
    seq_len, batch, d = x.shape
    return pl.pallas_call(
        _pipelined_kernel,
        in_specs=[
            pl.BlockSpec(memory_space=pl.ANY),
            pl.BlockSpec(memory_space=pl.ANY),
        ],
        out_specs=pl.BlockSpec(memory_space=pl.ANY),
        out_shape=jax.ShapeDtypeStruct((seq_len, batch, d), x.dtype),
        scratch_shapes=[
            pltpu.VMEM((_NBUF, _BS, batch, d), x.dtype),
            pltpu.VMEM((_TNBUF, _TG * _BS, d), table.dtype),
            pltpu.VMEM((_NBUF, _BS, batch, d), x.dtype),
            pltpu.SemaphoreType.DMA((_NBUF,)),
            pltpu.SemaphoreType.DMA((_TNBUF,)),
            pltpu.SemaphoreType.DMA((_NBUF,)),
        ],
    )(x, table)
